# Initial kernel scaffold; baseline (speedup 1.0000x reference)
import jax
import jax.numpy as jnp
from jax.experimental import pallas as pl


def kernel(x, edge_index, outfall_indices, W1, b1, W2, b2, Wl, bl):
    raise NotImplementedError("write your pallas kernel here")


# speedup vs baseline: 26.2493x; 26.2493x over previous
import functools

import jax
import jax.numpy as jnp
from jax import lax
from jax.experimental import pallas as pl
from jax.experimental.pallas import tpu as pltpu
from jax.experimental.pallas import tpu_sc as plsc

N = 10000
E = 320000
D_IN = 128
H = 48

NC = 2
NS = 16
NW = NC * NS
EPT = E // NW
CH = 80
NITER = EPT // CH
NP = 10240
RPS = NP // NS


@functools.lru_cache(maxsize=None)
def _make_sc_degree():
  mesh = plsc.VectorSubcoreMesh(
      core_axis_name="c", subcore_axis_name="s",
      num_cores=NC, num_subcores=NS)

  @functools.partial(
      pl.kernel,
      out_type=jax.ShapeDtypeStruct((NW, NP), jnp.float32),
      mesh=mesh,
      compiler_params=pltpu.CompilerParams(needs_layout_passes=False, use_tc_tiling_on_sc=False),
      scratch_types=[
          pltpu.VMEM((NITER, CH), jnp.int32),
          pltpu.VMEM((NP,), jnp.float32),
      ],
  )
  def _sc_degree(dst3_hbm, degp_hbm, dst_v, deg_v):
    cid = lax.axis_index("c")
    sid = lax.axis_index("s")
    wid = sid * NC + cid

    def zero_body(i, _):
        deg_v[pl.ds(i * 16, 16)] = jnp.zeros((16,), jnp.float32)
        return 0
    lax.fori_loop(0, NP // 16, zero_body, 0)

    pltpu.sync_copy(dst3_hbm.at[wid], dst_v)

    ones = jnp.ones((16,), jnp.float32)

    def body(j, _):
        for c in range(CH // 16):
            idx = dst_v[j, pl.ds(c * 16, 16)]
            plsc.addupdate_scatter(deg_v, [idx], ones)
        return 0
    lax.fori_loop(0, NITER, body, 0)

    pltpu.sync_copy(deg_v, degp_hbm.at[wid])

  return _sc_degree


def _sc_degree(dst3):
  return _make_sc_degree()(dst3)


@functools.lru_cache(maxsize=None)
def _make_sc_edge():
  mesh = plsc.VectorSubcoreMesh(
      core_axis_name="c", subcore_axis_name="s",
      num_cores=NC, num_subcores=NS)

  @functools.partial(
      pl.kernel,
      out_type=jax.ShapeDtypeStruct((NC, NP, H), jnp.float32),
      mesh=mesh,
      compiler_params=pltpu.CompilerParams(needs_layout_passes=False, use_tc_tiling_on_sc=False),
      scratch_types=[
          pltpu.VMEM((NITER, CH), jnp.int32),
          pltpu.VMEM((NITER, CH), jnp.int32),
          pltpu.VMEM((CH, H), jnp.float32),
          pltpu.VMEM_SHARED((NP, H), jnp.float32),
          pltpu.SemaphoreType.DMA,
      ],
  )
  def _sc_edge_k(src3_hbm, dst3_hbm, table_hbm, zeros_hbm, out_hbm,
                 src_v, dst_v, rows_v, acc_sh, sem):
    cid = lax.axis_index("c")
    sid = lax.axis_index("s")
    wid = sid * NC + cid

    row0 = pl.multiple_of(sid * RPS, 8)
    pltpu.sync_copy(zeros_hbm.at[pl.ds(row0, RPS)],
                    acc_sh.at[pl.ds(row0, RPS)])
    pltpu.sync_copy(src3_hbm.at[wid], src_v)
    pltpu.sync_copy(dst3_hbm.at[wid], dst_v)
    plsc.subcore_barrier()

    def body(j, _):
        pltpu.async_copy(table_hbm.at[src_v.at[j]], rows_v, sem).wait()
        pltpu.sync_copy(rows_v, acc_sh.at[dst_v.at[j]], add=True)
        return 0
    lax.fori_loop(0, NITER, body, 0)

    plsc.subcore_barrier()
    pltpu.sync_copy(acc_sh.at[pl.ds(row0, RPS)],
                    out_hbm.at[cid, pl.ds(row0, RPS)])

  return _sc_edge_k


def _sc_edge(src3, dst3, table, zeros):
  return _make_sc_edge()(src3, dst3, table, zeros)


def _tc1_body(degp_ref, x_ref, w1_ref, ptil_ref, dis_ref):
    i = pl.program_id(0)
    blk = x_ref.shape[0]
    deg = jnp.sum(degp_ref[:, pl.ds(i * blk, blk)], axis=0) + 1.0
    dis = lax.rsqrt(deg)
    p = jnp.dot(x_ref[...], w1_ref[...], preferred_element_type=jnp.float32)
    ptil_ref[...] = p * dis[:, None]
    dis_ref[...] = dis[:, None]


def _tc1(degp, x, w1):
    blk = 2048
    grid = NP // blk
    return pl.pallas_call(
        _tc1_body,
        grid=(grid,),
        in_specs=[
            pl.BlockSpec((NW, NP), lambda i: (0, 0)),
            pl.BlockSpec((blk, D_IN), lambda i: (i, 0)),
            pl.BlockSpec((D_IN, H), lambda i: (0, 0)),
        ],
        out_specs=[
            pl.BlockSpec((blk, H), lambda i: (i, 0)),
            pl.BlockSpec((blk, 1), lambda i: (i, 0)),
        ],
        out_shape=[
            jax.ShapeDtypeStruct((NP, H), jnp.float32),
            jax.ShapeDtypeStruct((NP, 1), jnp.float32),
        ],
    )(degp, x, w1)


def _tc2_body(s_ref, ptil_ref, dis_ref, w2_ref, b1_ref, qtil_ref):
    dis = dis_ref[...]
    agg = dis * (s_ref[0] + s_ref[1] + ptil_ref[...]) + b1_ref[...]
    h1 = jnp.maximum(agg, 0.0)
    q = jnp.dot(h1, w2_ref[...], preferred_element_type=jnp.float32)
    qtil_ref[...] = q * dis


def _tc2(s1, ptil, dis, w2, b1):
    blk = 2048
    grid = NP // blk
    return pl.pallas_call(
        _tc2_body,
        grid=(grid,),
        in_specs=[
            pl.BlockSpec((NC, blk, H), lambda i: (0, i, 0)),
            pl.BlockSpec((blk, H), lambda i: (i, 0)),
            pl.BlockSpec((blk, 1), lambda i: (i, 0)),
            pl.BlockSpec((H, H), lambda i: (0, 0)),
            pl.BlockSpec((1, H), lambda i: (0, 0)),
        ],
        out_specs=pl.BlockSpec((blk, H), lambda i: (i, 0)),
        out_shape=jax.ShapeDtypeStruct((NP, H), jnp.float32),
    )(s1, ptil, dis, w2, b1)


def _tc3_body(outf_ref, s_ref, qtil_ref, dis_ref, b2_ref, wl_ref, bl_ref,
              out_ref, rows_ref):
    rows_ref[...] = jnp.zeros((16, H), jnp.float32)
    for j in range(9):
        idx = outf_ref[j]
        srow = (s_ref[0, pl.ds(idx, 1), :] + s_ref[1, pl.ds(idx, 1), :]
                + qtil_ref[pl.ds(idx, 1), :])
        d = dis_ref[pl.ds(idx, 1), :]
        h2 = jnp.maximum(d * srow + b2_ref[...], 0.0)
        rows_ref[pl.ds(j, 1), :] = h2
    z = jnp.dot(rows_ref[...], wl_ref[...], preferred_element_type=jnp.float32)
    out_ref[...] = jax.nn.sigmoid(z + bl_ref[...])


def _tc3(outfall, s2, qtil, dis, b2, wl, bl):
    return pl.pallas_call(
        _tc3_body,
        in_specs=[
            pl.BlockSpec(memory_space=pltpu.SMEM),
            pl.BlockSpec((NC, NP, H), lambda: (0, 0, 0)),
            pl.BlockSpec((NP, H), lambda: (0, 0)),
            pl.BlockSpec((NP, 1), lambda: (0, 0)),
            pl.BlockSpec((1, H), lambda: (0, 0)),
            pl.BlockSpec((H, 1), lambda: (0, 0)),
            pl.BlockSpec((1, 1), lambda: (0, 0)),
        ],
        out_specs=pl.BlockSpec((16, 1), lambda: (0, 0)),
        out_shape=jax.ShapeDtypeStruct((16, 1), jnp.float32),
        scratch_shapes=[pltpu.VMEM((16, H), jnp.float32)],
    )(outfall, s2, qtil, dis, b2, wl, bl)


def kernel(x, edge_index, outfall_indices, W1, b1, W2, b2, Wl, bl):
    src3 = edge_index[0].reshape(NW, NITER, CH)
    dst3 = edge_index[1].reshape(NW, NITER, CH)
    zeros = jnp.zeros((NP, H), jnp.float32)
    x = jnp.concatenate([x, jnp.zeros((NP - N, D_IN), jnp.float32)], axis=0)

    degp = _sc_degree(dst3)
    ptil, dis = _tc1(degp, x, W1)
    s1 = _sc_edge(src3, dst3, ptil, zeros)
    qtil = _tc2(s1, ptil, dis, W2, b1.reshape(1, H))
    s2 = _sc_edge(src3, dst3, qtil, zeros)
    out = _tc3(outfall_indices, s2, qtil, dis, b2.reshape(1, H),
               Wl, bl.reshape(1, 1))
    return out[:9, 0]
